# baseline (speedup 1.0000x reference)
import functools

import jax
import jax.numpy as jnp
from jax.experimental import pallas as pl
from jax.experimental.pallas import tpu as pltpu

_EPS = 1e-12
_ATOL = 1e-8
_RTOL = 1e-5
_KCHUNK = 512


def _row_ss(x):
    return jnp.sum(x * x, axis=1, keepdims=True)


def _normalize_rows(x):
    return x / jnp.maximum(jnp.sqrt(_row_ss(x)), _EPS)


def _rank_kernel(q_ref, t_ref, d0_ref, d1_ref, out_ref, qn_ref, te_ref, cnt_ref):
    k = pl.program_id(0)

    @pl.when(k == 0)
    def _init():
        qn = _normalize_rows(q_ref[...])
        tn = _normalize_rows(t_ref[...])
        qn_ref[...] = qn
        m = jax.lax.dot_general(qn, tn, (((1,), (1,)), ((), ())))
        nq = m.shape[0]
        eye = (jax.lax.broadcasted_iota(jnp.int32, (nq, nq), 0)
               == jax.lax.broadcasted_iota(jnp.int32, (nq, nq), 1))
        thr = jnp.sum(jnp.where(eye, m, 0.0), axis=0, keepdims=True)
        te_ref[...] = thr - (_ATOL + _RTOL * jnp.abs(thr))

    cnt = None
    for d_ref in (d0_ref, d1_ref):
        dim = d_ref.shape[1]
        s = None
        ss = None
        for c in range(0, dim, _KCHUNK):
            dc = d_ref[:, c:c + _KCHUNK]
            qc = qn_ref[:, c:c + _KCHUNK]
            ps = jax.lax.dot_general(dc, qc, (((1,), (1,)), ((), ())))
            pss = _row_ss(dc)
            s = ps if s is None else s + ps
            ss = pss if ss is None else ss + pss
        norm = jnp.maximum(jnp.sqrt(ss), _EPS)
        ge = s >= te_ref[...] * norm
        pc = jnp.sum(ge.astype(jnp.int32), axis=0, keepdims=True)
        cnt = pc if cnt is None else cnt + pc

    @pl.when(k == 0)
    def _first():
        cnt_ref[...] = cnt - 1

    @pl.when(k != 0)
    def _rest():
        cnt_ref[...] = cnt_ref[...] + cnt

    @pl.when(k == pl.num_programs(0) - 1)
    def _emit():
        out_ref[...] = cnt_ref[...]


@functools.partial(jax.jit, static_argnames=("block",))
def _rank(queries, truths, data, block=512):
    n, d = data.shape
    nq = queries.shape[0]
    return pl.pallas_call(
        _rank_kernel,
        grid=(n // block,),
        in_specs=[
            pl.BlockSpec((nq, d), lambda k: (0, 0)),
            pl.BlockSpec((nq, d), lambda k: (0, 0)),
            pl.BlockSpec((block // 2, d), lambda k: (2 * k, 0)),
            pl.BlockSpec((block // 2, d), lambda k: (2 * k + 1, 0)),
        ],
        out_specs=pl.BlockSpec((1, nq), lambda k: (0, 0)),
        out_shape=jax.ShapeDtypeStruct((1, nq), jnp.int32),
        scratch_shapes=[
            pltpu.VMEM((nq, d), jnp.float32),
            pltpu.VMEM((1, nq), jnp.float32),
            pltpu.VMEM((1, nq), jnp.int32),
        ],
        compiler_params=pltpu.CompilerParams(
            dimension_semantics=("arbitrary",),
        ),
    )(queries, truths, data, data)


def kernel(queries, truths, data):
    return _rank(queries, truths, data)
